# Initial kernel scaffold; baseline (speedup 1.0000x reference)
import jax
import jax.numpy as jnp
from jax.experimental import pallas as pl


def kernel(x, edge_index, W1, as1, ad1, b1, W2, as2, ad2, b2, W3, as3, ad3, b3, Wc1, bc1, Wc2, bc2):
    raise NotImplementedError("write your pallas kernel here")


# speedup vs baseline: 59.2600x; 59.2600x over previous
import functools

import jax
import jax.numpy as jnp
import numpy as np
from jax import lax
from jax.experimental import pallas as pl
from jax.experimental.pallas import tpu as pltpu
from jax.experimental.pallas import tpu_sc as plsc

NN = 10000
DD = 128
NHEADS = 8
NHID = 16
NP = 10240
NCORES = 2
NSUB = 16
NWORK = NCORES * NSUB
CHUNK = 128
NEDGE = 320000
EE = NEDGE + NN
EP = ((EE + NWORK * CHUNK - 1) // (NWORK * CHUNK)) * (NWORK * CHUNK)
EW = EP // NWORK
NCH = EW // CHUNK
STRIPE = NP // NSUB
SB = STRIPE // CHUNK

_GD = lax.GatherDimensionNumbers(
    offset_dims=(), collapsed_slice_dims=(0,), start_index_map=(0,))


def _splat(vec, i):
    idx = jnp.full((16, 1), i, jnp.int32)
    return lax.gather(vec, idx, _GD, (1,),
                      mode=lax.GatherScatterMode.PROMISE_IN_BOUNDS)


def _rot8():
    return ((lax.iota(jnp.int32, 16) & 7) + 8).reshape(16, 1)


def _edge_pass(roww, heads):
    mesh = plsc.VectorSubcoreMesh(
        core_axis_name="c", subcore_axis_name="s",
        num_cores=NCORES, num_subcores=NSUB)

    @functools.partial(
        pl.kernel,
        out_type=(
            jax.ShapeDtypeStruct((NCORES, NP, roww), jnp.float32),
            jax.ShapeDtypeStruct((NCORES, NP, 16), jnp.float32),
        ),
        mesh=mesh,
        compiler_params=pltpu.CompilerParams(use_tc_tiling_on_sc=False),
        scratch_types=[
            pltpu.VMEM_SHARED((NP, roww), jnp.float32),
            pltpu.VMEM_SHARED((NP, 16), jnp.float32),
            pltpu.VMEM_SHARED((NP, 16), jnp.float32),
            pltpu.VMEM((CHUNK,), jnp.int32),
            pltpu.VMEM((CHUNK,), jnp.int32),
            pltpu.VMEM((CHUNK, 16), jnp.float32),
            pltpu.VMEM((CHUNK, 16), jnp.float32),
            pltpu.VMEM((CHUNK, 16), jnp.float32),
            pltpu.VMEM((CHUNK, roww), jnp.float32),
        ],
    )
    def k(h_hbm, st_hbm, src_hbm, dst_hbm,
          acc_out, es_out, acc_sh, es_sh, st_sh,
          srcv, dstv, av, bv, exv, hr):
        c = lax.axis_index("c")
        s = lax.axis_index("s")
        w = c * NSUB + s
        z16 = jnp.zeros((16,), jnp.float32)

        def zrow(e, cc):
            for kk in range(roww // 16):
                hr[e, pl.ds(kk * 16, 16)] = z16
            exv[e] = z16
            return cc

        lax.fori_loop(0, CHUNK, zrow, 0)

        def stage(j, cc):
            rows = pl.ds(s * STRIPE + j * CHUNK, CHUNK)
            pltpu.sync_copy(hr, acc_sh.at[rows])
            pltpu.sync_copy(exv, es_sh.at[rows])
            pltpu.sync_copy(st_hbm.at[rows], av)
            pltpu.sync_copy(av, st_sh.at[rows])
            return cc

        lax.fori_loop(0, SB, stage, 0)
        plsc.subcore_barrier()

        cbase = w * EW
        rot8 = _rot8()

        def chunk_body(i, carry):
            base = cbase + i * CHUNK
            pltpu.sync_copy(src_hbm.at[pl.ds(base, CHUNK)], srcv)
            pltpu.sync_copy(dst_hbm.at[pl.ds(base, CHUNK)], dstv)
            pltpu.sync_copy(st_sh.at[srcv], av)
            pltpu.sync_copy(st_sh.at[dstv], bv)
            pltpu.sync_copy(h_hbm.at[srcv], hr)

            def edge_body(e, ecarry):
                a = av[e]
                b = bv[e]
                brot = lax.gather(b, rot8, _GD, (1,),
                                  mode=lax.GatherScatterMode.PROMISE_IN_BOUNDS)
                al = a + brot
                al = jnp.maximum(al, 0.2 * al)
                ex = jnp.exp(al)
                exv[e] = ex
                for hd in range(heads):
                    sp = _splat(ex, hd)
                    cols = pl.ds(hd * NHID, NHID)
                    hr[e, cols] = hr[e, cols] * sp
                return ecarry

            lax.fori_loop(0, CHUNK, edge_body, 0)
            pltpu.sync_copy(exv, es_sh.at[dstv], add=True)
            pltpu.sync_copy(hr, acc_sh.at[dstv], add=True)
            return carry

        lax.fori_loop(0, NCH, chunk_body, 0)
        plsc.subcore_barrier()

        def unstage(j, cc):
            rows = pl.ds(s * STRIPE + j * CHUNK, CHUNK)
            pltpu.sync_copy(acc_sh.at[rows], hr)
            pltpu.sync_copy(hr, acc_out.at[c, rows])
            pltpu.sync_copy(es_sh.at[rows], exv)
            pltpu.sync_copy(exv, es_out.at[c, rows])
            return cc

        lax.fori_loop(0, SB, unstage, 0)

    return k


_edge128 = _edge_pass(NHEADS * NHID, NHEADS)
_edge16 = _edge_pass(NHID, 1)

_BR = 1024
_RB = NP // _BR


def _tc_first(xp, Wt, Ast, dout):
    def body(x_ref, w_ref, a_ref, h_ref, st_ref):
        h = jnp.dot(x_ref[...], w_ref[...], preferred_element_type=jnp.float32)
        h_ref[...] = h
        st_ref[...] = jnp.dot(h, a_ref[...], preferred_element_type=jnp.float32)

    din = xp.shape[1]
    return pl.pallas_call(
        body,
        grid=(_RB,),
        in_specs=[
            pl.BlockSpec((_BR, din), lambda i: (i, 0)),
            pl.BlockSpec((din, dout), lambda i: (0, 0)),
            pl.BlockSpec((dout, 16), lambda i: (0, 0)),
        ],
        out_specs=[
            pl.BlockSpec((_BR, dout), lambda i: (i, 0)),
            pl.BlockSpec((_BR, 16), lambda i: (i, 0)),
        ],
        out_shape=[
            jax.ShapeDtypeStruct((NP, dout), jnp.float32),
            jax.ShapeDtypeStruct((NP, 16), jnp.float32),
        ],
    )(xp, Wt, Ast)


def _tc_mid(a0, a1, es0, es1, Emat, b, Wt, Ast, dout):
    dprev = a0.shape[1]

    def body(a0_ref, a1_ref, e0_ref, e1_ref, em_ref, b_ref, w_ref,
             a_ref, h_ref, st_ref):
        accs = a0_ref[...] + a1_ref[...]
        es = e0_ref[...] + e1_ref[...]
        den = jnp.dot(es, em_ref[...], preferred_element_type=jnp.float32)
        v = jnp.where(den > 0.0, accs / den, 0.0) + b_ref[...]
        xin = jnp.where(v > 0.0, v, jnp.exp(v) - 1.0)
        h = jnp.dot(xin, w_ref[...], preferred_element_type=jnp.float32)
        h_ref[...] = h
        st_ref[...] = jnp.dot(h, a_ref[...], preferred_element_type=jnp.float32)

    return pl.pallas_call(
        body,
        grid=(_RB,),
        in_specs=[
            pl.BlockSpec((_BR, dprev), lambda i: (i, 0)),
            pl.BlockSpec((_BR, dprev), lambda i: (i, 0)),
            pl.BlockSpec((_BR, 16), lambda i: (i, 0)),
            pl.BlockSpec((_BR, 16), lambda i: (i, 0)),
            pl.BlockSpec((16, dprev), lambda i: (0, 0)),
            pl.BlockSpec((1, dprev), lambda i: (0, 0)),
            pl.BlockSpec((dprev, dout), lambda i: (0, 0)),
            pl.BlockSpec((dout, 16), lambda i: (0, 0)),
        ],
        out_specs=[
            pl.BlockSpec((_BR, dout), lambda i: (i, 0)),
            pl.BlockSpec((_BR, 16), lambda i: (i, 0)),
        ],
        out_shape=[
            jax.ShapeDtypeStruct((NP, dout), jnp.float32),
            jax.ShapeDtypeStruct((NP, 16), jnp.float32),
        ],
    )(a0, a1, es0, es1, Emat, b, Wt, Ast)


def _tc_final(a0, a1, es0, es1, b3, Wc1, bc1, Wc2, bc2):
    def body(a0_ref, a1_ref, e0_ref, e1_ref, b_ref, w1_ref, b1_ref,
             w2_ref, b2_ref, o_ref):
        accs = a0_ref[...] + a1_ref[...]
        den = (e0_ref[...] + e1_ref[...])[:, 0:1]
        h3 = jnp.where(den > 0.0, accs / den, 0.0) + b_ref[...]
        ridx = lax.broadcasted_iota(jnp.int32, (NP, NHID), 0)
        h3 = jnp.where(ridx < NN, h3, 0.0)
        g = jnp.sum(h3, axis=0, keepdims=True) * (1.0 / NN)
        z = jnp.sum(g * w1_ref[...], axis=1, keepdims=True)
        z = jnp.maximum(z.T + b1_ref[...], 0.0)
        o = jnp.sum(z * w2_ref[...], axis=1, keepdims=True)
        o_ref[...] = o.T + b2_ref[...]

    return pl.pallas_call(
        body,
        out_shape=jax.ShapeDtypeStruct((1, 10), jnp.float32),
    )(a0, a1, es0, es1, b3, Wc1, bc1, Wc2, bc2)


def kernel(x, edge_index, W1, as1, ad1, b1, W2, as2, ad2, b2,
           W3, as3, ad3, b3, Wc1, bc1, Wc2, bc2):
    f32 = jnp.float32
    xp = jnp.zeros((NP, DD), f32).at[:NN].set(x)
    loop = jnp.arange(NN, dtype=jnp.int32)
    src = jnp.concatenate([edge_index[0].astype(jnp.int32), loop])
    dst = jnp.concatenate([edge_index[1].astype(jnp.int32), loop])
    fill = jnp.full((EP - EE,), NN, jnp.int32)
    src = jnp.concatenate([src, fill])
    dst = jnp.concatenate([dst, fill])

    eye8 = jnp.eye(NHEADS, dtype=f32)

    def attmat(a):
        return (a[0][:, :, None] * eye8[:, None, :]).reshape(
            NHEADS * NHID, NHEADS)

    Ast1 = jnp.concatenate([attmat(as1), attmat(ad1)], axis=1)
    Ast2 = jnp.concatenate([attmat(as2), attmat(ad2)], axis=1)
    z7 = jnp.zeros((NHID, 7), f32)
    Ast3 = jnp.concatenate(
        [as3[0, 0][:, None], z7, ad3[0, 0][:, None], z7], axis=1)
    Emat8 = jnp.pad(jnp.repeat(eye8, NHID, axis=1), ((0, 8), (0, 0)))

    h1, ST1 = _tc_first(xp, W1.T, Ast1, NHEADS * NHID)
    acc1, es1 = _edge128(h1, ST1, src, dst)
    h2, ST2 = _tc_mid(acc1[0], acc1[1], es1[0], es1[1], Emat8,
                      b1[None, :], W2.T, Ast2, NHEADS * NHID)
    acc2, es2 = _edge128(h2, ST2, src, dst)
    h3, ST3 = _tc_mid(acc2[0], acc2[1], es2[0], es2[1], Emat8,
                      b2[None, :], W3.T, Ast3, NHID)
    acc3, es3 = _edge16(h3, ST3, src, dst)
    return _tc_final(acc3[0], acc3[1], es3[0], es3[1],
                     b3[None, :], Wc1, bc1[None, :], Wc2, bc2[None, :])


# speedup vs baseline: 76.1941x; 1.2858x over previous
import functools

import jax
import jax.numpy as jnp
import numpy as np
from jax import lax
from jax.experimental import pallas as pl
from jax.experimental.pallas import tpu as pltpu
from jax.experimental.pallas import tpu_sc as plsc

NN = 10000
DD = 128
NHEADS = 8
NHID = 16
NP = 10240
NCORES = 2
NSUB = 16
NWORK = NCORES * NSUB
CHUNK = 128
NEDGE = 320000
EE = NEDGE + NN
EP = ((EE + NWORK * CHUNK - 1) // (NWORK * CHUNK)) * (NWORK * CHUNK)
EW = EP // NWORK
NCH = EW // CHUNK
STRIPE = NP // NSUB
SB = STRIPE // CHUNK

_GD = lax.GatherDimensionNumbers(
    offset_dims=(), collapsed_slice_dims=(0,), start_index_map=(0,))


def _splat(vec, i):
    idx = jnp.full((16, 1), i, jnp.int32)
    return lax.gather(vec, idx, _GD, (1,),
                      mode=lax.GatherScatterMode.PROMISE_IN_BOUNDS)


def _rot8():
    return ((lax.iota(jnp.int32, 16) & 7) + 8).reshape(16, 1)


def _edge_pass(roww, heads):
    mesh = plsc.VectorSubcoreMesh(
        core_axis_name="c", subcore_axis_name="s",
        num_cores=NCORES, num_subcores=NSUB)

    @functools.partial(
        pl.kernel,
        out_type=(
            jax.ShapeDtypeStruct((NCORES, NP, roww), jnp.float32),
            jax.ShapeDtypeStruct((NCORES, NP, 16), jnp.float32),
        ),
        mesh=mesh,
        compiler_params=pltpu.CompilerParams(use_tc_tiling_on_sc=False),
        scratch_types=[
            pltpu.VMEM_SHARED((NP, roww), jnp.float32),
            pltpu.VMEM_SHARED((NP, 16), jnp.float32),
            pltpu.VMEM_SHARED((NP, 16), jnp.float32),
            pltpu.VMEM((CHUNK,), jnp.int32),
            pltpu.VMEM((CHUNK,), jnp.int32),
            pltpu.VMEM((CHUNK, 16), jnp.float32),
            pltpu.VMEM((CHUNK, 16), jnp.float32),
            pltpu.VMEM((CHUNK, 16), jnp.float32),
            pltpu.VMEM((CHUNK, roww), jnp.float32),
        ],
    )
    def k(h_hbm, st_hbm, src_hbm, dst_hbm,
          acc_out, es_out, acc_sh, es_sh, st_sh,
          srcv, dstv, av, bv, exv, hr):
        c = lax.axis_index("c")
        s = lax.axis_index("s")
        w = c * NSUB + s
        z16 = jnp.zeros((16,), jnp.float32)

        def zrow(e, cc):
            for kk in range(roww // 16):
                hr[e, pl.ds(kk * 16, 16)] = z16
            exv[e] = z16
            return cc

        lax.fori_loop(0, CHUNK, zrow, 0)

        def stage(j, cc):
            rows = pl.ds(s * STRIPE + j * CHUNK, CHUNK)
            pltpu.sync_copy(hr, acc_sh.at[rows])
            pltpu.sync_copy(exv, es_sh.at[rows])
            pltpu.sync_copy(st_hbm.at[rows], av)
            pltpu.sync_copy(av, st_sh.at[rows])
            return cc

        lax.fori_loop(0, SB, stage, 0)
        plsc.subcore_barrier()

        cbase = w * EW
        rot8 = _rot8()

        def chunk_body(i, carry):
            base = cbase + i * CHUNK
            pltpu.sync_copy(src_hbm.at[pl.ds(base, CHUNK)], srcv)
            pltpu.sync_copy(dst_hbm.at[pl.ds(base, CHUNK)], dstv)
            pltpu.sync_copy(st_sh.at[srcv], av)
            pltpu.sync_copy(st_sh.at[dstv], bv)
            pltpu.sync_copy(h_hbm.at[srcv], hr)

            @plsc.parallel_loop(0, CHUNK, unroll=4)
            def edge_body(e):
                a = av[e]
                b = bv[e]
                brot = lax.gather(
                    b, rot8, _GD, (1,),
                    mode=lax.GatherScatterMode.PROMISE_IN_BOUNDS)
                al = a + brot
                al = jnp.maximum(al, 0.2 * al)
                ex = jnp.exp(al)
                exv[e] = ex
                rows = [hr[e, pl.ds(hd * NHID, NHID)] for hd in range(heads)]
                for hd in range(heads):
                    hr[e, pl.ds(hd * NHID, NHID)] = rows[hd] * _splat(ex, hd)
            pltpu.sync_copy(exv, es_sh.at[dstv], add=True)
            pltpu.sync_copy(hr, acc_sh.at[dstv], add=True)
            return carry

        lax.fori_loop(0, NCH, chunk_body, 0)
        plsc.subcore_barrier()

        def unstage(j, cc):
            rows = pl.ds(s * STRIPE + j * CHUNK, CHUNK)
            pltpu.sync_copy(acc_sh.at[rows], hr)
            pltpu.sync_copy(hr, acc_out.at[c, rows])
            pltpu.sync_copy(es_sh.at[rows], exv)
            pltpu.sync_copy(exv, es_out.at[c, rows])
            return cc

        lax.fori_loop(0, SB, unstage, 0)

    return k


_edge128 = _edge_pass(NHEADS * NHID, NHEADS)
_edge16 = _edge_pass(NHID, 1)

_BR = 1024
_RB = NP // _BR


def _tc_first(xp, Wt, Ast, dout):
    def body(x_ref, w_ref, a_ref, h_ref, st_ref):
        h = jnp.dot(x_ref[...], w_ref[...], preferred_element_type=jnp.float32)
        h_ref[...] = h
        st_ref[...] = jnp.dot(h, a_ref[...], preferred_element_type=jnp.float32)

    din = xp.shape[1]
    return pl.pallas_call(
        body,
        grid=(_RB,),
        in_specs=[
            pl.BlockSpec((_BR, din), lambda i: (i, 0)),
            pl.BlockSpec((din, dout), lambda i: (0, 0)),
            pl.BlockSpec((dout, 16), lambda i: (0, 0)),
        ],
        out_specs=[
            pl.BlockSpec((_BR, dout), lambda i: (i, 0)),
            pl.BlockSpec((_BR, 16), lambda i: (i, 0)),
        ],
        out_shape=[
            jax.ShapeDtypeStruct((NP, dout), jnp.float32),
            jax.ShapeDtypeStruct((NP, 16), jnp.float32),
        ],
    )(xp, Wt, Ast)


def _tc_mid(a0, a1, es0, es1, Emat, b, Wt, Ast, dout):
    dprev = a0.shape[1]

    def body(a0_ref, a1_ref, e0_ref, e1_ref, em_ref, b_ref, w_ref,
             a_ref, h_ref, st_ref):
        accs = a0_ref[...] + a1_ref[...]
        es = e0_ref[...] + e1_ref[...]
        den = jnp.dot(es, em_ref[...], preferred_element_type=jnp.float32)
        v = jnp.where(den > 0.0, accs / den, 0.0) + b_ref[...]
        xin = jnp.where(v > 0.0, v, jnp.exp(v) - 1.0)
        h = jnp.dot(xin, w_ref[...], preferred_element_type=jnp.float32)
        h_ref[...] = h
        st_ref[...] = jnp.dot(h, a_ref[...], preferred_element_type=jnp.float32)

    return pl.pallas_call(
        body,
        grid=(_RB,),
        in_specs=[
            pl.BlockSpec((_BR, dprev), lambda i: (i, 0)),
            pl.BlockSpec((_BR, dprev), lambda i: (i, 0)),
            pl.BlockSpec((_BR, 16), lambda i: (i, 0)),
            pl.BlockSpec((_BR, 16), lambda i: (i, 0)),
            pl.BlockSpec((16, dprev), lambda i: (0, 0)),
            pl.BlockSpec((1, dprev), lambda i: (0, 0)),
            pl.BlockSpec((dprev, dout), lambda i: (0, 0)),
            pl.BlockSpec((dout, 16), lambda i: (0, 0)),
        ],
        out_specs=[
            pl.BlockSpec((_BR, dout), lambda i: (i, 0)),
            pl.BlockSpec((_BR, 16), lambda i: (i, 0)),
        ],
        out_shape=[
            jax.ShapeDtypeStruct((NP, dout), jnp.float32),
            jax.ShapeDtypeStruct((NP, 16), jnp.float32),
        ],
    )(a0, a1, es0, es1, Emat, b, Wt, Ast)


def _tc_final(a0, a1, es0, es1, b3, Wc1, bc1, Wc2, bc2):
    def body(a0_ref, a1_ref, e0_ref, e1_ref, b_ref, w1_ref, b1_ref,
             w2_ref, b2_ref, o_ref):
        accs = a0_ref[...] + a1_ref[...]
        den = (e0_ref[...] + e1_ref[...])[:, 0:1]
        h3 = jnp.where(den > 0.0, accs / den, 0.0) + b_ref[...]
        ridx = lax.broadcasted_iota(jnp.int32, (NP, NHID), 0)
        h3 = jnp.where(ridx < NN, h3, 0.0)
        g = jnp.sum(h3, axis=0, keepdims=True) * (1.0 / NN)
        z = jnp.sum(g * w1_ref[...], axis=1, keepdims=True)
        z = jnp.maximum(z.T + b1_ref[...], 0.0)
        o = jnp.sum(z * w2_ref[...], axis=1, keepdims=True)
        o_ref[...] = o.T + b2_ref[...]

    return pl.pallas_call(
        body,
        out_shape=jax.ShapeDtypeStruct((1, 10), jnp.float32),
    )(a0, a1, es0, es1, b3, Wc1, bc1, Wc2, bc2)


def kernel(x, edge_index, W1, as1, ad1, b1, W2, as2, ad2, b2,
           W3, as3, ad3, b3, Wc1, bc1, Wc2, bc2):
    f32 = jnp.float32
    xp = jnp.zeros((NP, DD), f32).at[:NN].set(x)
    loop = jnp.arange(NN, dtype=jnp.int32)
    src = jnp.concatenate([edge_index[0].astype(jnp.int32), loop])
    dst = jnp.concatenate([edge_index[1].astype(jnp.int32), loop])
    fill = jnp.full((EP - EE,), NN, jnp.int32)
    src = jnp.concatenate([src, fill])
    dst = jnp.concatenate([dst, fill])

    eye8 = jnp.eye(NHEADS, dtype=f32)

    def attmat(a):
        return (a[0][:, :, None] * eye8[:, None, :]).reshape(
            NHEADS * NHID, NHEADS)

    Ast1 = jnp.concatenate([attmat(as1), attmat(ad1)], axis=1)
    Ast2 = jnp.concatenate([attmat(as2), attmat(ad2)], axis=1)
    z7 = jnp.zeros((NHID, 7), f32)
    Ast3 = jnp.concatenate(
        [as3[0, 0][:, None], z7, ad3[0, 0][:, None], z7], axis=1)
    Emat8 = jnp.pad(jnp.repeat(eye8, NHID, axis=1), ((0, 8), (0, 0)))

    h1, ST1 = _tc_first(xp, W1.T, Ast1, NHEADS * NHID)
    acc1, es1 = _edge128(h1, ST1, src, dst)
    h2, ST2 = _tc_mid(acc1[0], acc1[1], es1[0], es1[1], Emat8,
                      b1[None, :], W2.T, Ast2, NHEADS * NHID)
    acc2, es2 = _edge128(h2, ST2, src, dst)
    h3, ST3 = _tc_mid(acc2[0], acc2[1], es2[0], es2[1], Emat8,
                      b2[None, :], W3.T, Ast3, NHID)
    acc3, es3 = _edge16(h3, ST3, src, dst)
    return _tc_final(acc3[0], acc3[1], es3[0], es3[1],
                     b3[None, :], Wc1, bc1[None, :], Wc2, bc2[None, :])
